# baseline (speedup 1.0000x reference)
import functools

import jax
import jax.numpy as jnp
from jax import lax
from jax.experimental import pallas as pl
from jax.experimental.pallas import tpu as pltpu
from jax.experimental.pallas import tpu_sc as plsc

_L = 16
_NC = 2
_NS = 16


def _bilinear_gather(bev, kp_flat, consts, B, C, N, H, W):
    NW = _NC * _NS
    per_w = (B * C) // NW
    w_per_b = NW // B
    n_grp = N // _L
    mesh = plsc.VectorSubcoreMesh(
        core_axis_name="c", subcore_axis_name="s",
        num_cores=_NC, num_subcores=_NS)

    @functools.partial(
        pl.kernel,
        out_type=jax.ShapeDtypeStruct((B, C, N), jnp.float32),
        mesh=mesh,
        compiler_params=pltpu.CompilerParams(
            needs_layout_passes=False, use_tc_tiling_on_sc=True),
        scratch_types=[
            pltpu.VMEM((N * 3,), jnp.float32),
            pltpu.VMEM((2, _L), jnp.float32),
            pltpu.VMEM((3, N), jnp.int32),
            pltpu.VMEM((4, N), jnp.float32),
            pltpu.VMEM((H, W), jnp.float32),
            pltpu.VMEM((N,), jnp.float32),
            pltpu.VMEM((N,), jnp.float32),
            pltpu.SemaphoreType.DMA,
            pltpu.SemaphoreType.DMA,
            pltpu.SemaphoreType.DMA,
        ],
    )
    def k(bev_hbm, kp_hbm, consts_hbm, out_hbm, kp_v, consts_v, idx_v, wgt_v,
          img, out0, out1, si, so0, so1):
        wid = lax.axis_index("s") * _NC + lax.axis_index("c")
        b = wid // w_per_b
        c0 = (wid % w_per_b) * per_w

        pltpu.async_copy(bev_hbm.at[b, c0], img, si)
        pltpu.sync_copy(kp_hbm.at[b], kp_v)
        pltpu.sync_copy(consts_hbm, consts_v)
        stride_v = consts_v[0, :]
        scale_v = consts_v[1, :]
        lane3 = lax.iota(jnp.int32, _L) * 3

        def prep_body(g2, carry):
            for u in range(2):
                g = g2 * 2 + u
                pos = lane3 + g * (3 * _L)
                xs = plsc.load_gather(kp_v, [pos])
                ys = plsc.load_gather(kp_v, [pos + 1])
                x = (xs - 0.0) / jnp.float32(0.05) / stride_v
                y = (ys - jnp.float32(-40.0)) / jnp.float32(0.05) / stride_v
                x0t = x.astype(jnp.int32)
                y0t = y.astype(jnp.int32)
                x0c = jnp.clip(x0t, 0, W - 1)
                x1c = jnp.clip(x0t + 1, 0, W - 1)
                y0c = jnp.clip(y0t, 0, H - 1)
                y1c = jnp.clip(y0t + 1, 0, H - 1)
                x0f = x0c.astype(jnp.float32)
                x1f = x1c.astype(jnp.float32)
                y0f = y0c.astype(jnp.float32)
                y1f = y1c.astype(jnp.float32)
                gx = x1f - x
                fx = x - x0f
                gy = y1f - y
                fy = y - y0f
                s = pl.ds(g * _L, _L)
                idx_v[0, s] = y0c
                idx_v[1, s] = x0c
                idx_v[2, s] = (y1c - y0c) * 2 + (x1c - x0c)
                wgt_v[0, s] = gx * gy * scale_v
                wgt_v[1, s] = gx * fy * scale_v
                wgt_v[2, s] = fx * gy * scale_v
                wgt_v[3, s] = fx * fy * scale_v
            return carry

        lax.fori_loop(0, n_grp // 2, prep_body, 0)

        obufs = ((out0, so0), (out1, so1))

        def chan_body(sidx, carry):
            for u in range(2):
                outb, so = obufs[u]
                c = c0 + sidx * 2 + u
                pltpu.make_async_copy(bev_hbm.at[b, c], img, si).wait()

                @pl.when(sidx * 2 + u >= 2)
                def _wait_prev_store():
                    pltpu.make_async_copy(outb, out_hbm.at[b, c], so).wait()

                def grp_body(g4, carry2):
                    for v in range(4):
                        g = g4 * 4 + v
                        s = pl.ds(g * _L, _L)
                        iy0 = idx_v[0, s]
                        ix0 = idx_v[1, s]
                        code = idx_v[2, s]
                        dy = code >> 1
                        dx = code & 1
                        iy1 = iy0 + dy
                        ix1 = ix0 + dx
                        acc = plsc.load_gather(img, [iy0, ix0]) * wgt_v[0, s]
                        acc = acc + plsc.load_gather(img, [iy1, ix0]) * wgt_v[1, s]
                        acc = acc + plsc.load_gather(img, [iy0, ix1]) * wgt_v[2, s]
                        acc = acc + plsc.load_gather(img, [iy1, ix1]) * wgt_v[3, s]
                        outb[s] = acc
                    return carry2

                lax.fori_loop(0, n_grp // 4, grp_body, 0)

                @pl.when(c + 1 < c0 + per_w)
                def _prefetch_next():
                    pltpu.async_copy(bev_hbm.at[b, c + 1], img, si)

                pltpu.async_copy(outb, out_hbm.at[b, c], so)
            return carry

        lax.fori_loop(0, per_w // 2, chan_body, 0)
        pltpu.make_async_copy(out0, out_hbm.at[b, c0], so0).wait()
        pltpu.make_async_copy(out1, out_hbm.at[b, c0], so1).wait()

    return k(bev, kp_flat, consts)


def kernel(keypoints, bev_features, batch_size, bev_stride):
    B, N, _ = keypoints.shape
    _, C, H, W = bev_features.shape
    stride_f = jnp.asarray(bev_stride, jnp.float32)
    scale_f = jnp.asarray(batch_size, jnp.float32) / B
    consts = jnp.stack([jnp.full((_L,), 1.0, jnp.float32) * stride_f,
                        jnp.full((_L,), 1.0, jnp.float32) * scale_f])
    kp_flat = keypoints.reshape(B, N * 3)
    out_bcn = _bilinear_gather(bev_features, kp_flat, consts, B, C, N, H, W)
    out = jnp.transpose(out_bcn, (0, 2, 1))
    return out


# speedup vs baseline: 1.2390x; 1.2390x over previous
import functools

import jax
import jax.numpy as jnp
from jax import lax
from jax.experimental import pallas as pl
from jax.experimental.pallas import tpu as pltpu
from jax.experimental.pallas import tpu_sc as plsc

_L = 16
_NC = 2
_NS = 16


def _bilinear_gather(bev2d, kp_flat, consts, B, C, N, H, W):
    NW = _NC * _NS
    per_w = (B * C) // NW
    w_per_b = NW // B
    n_grp = N // _L
    mesh = plsc.VectorSubcoreMesh(
        core_axis_name="c", subcore_axis_name="s",
        num_cores=_NC, num_subcores=_NS)

    @functools.partial(
        pl.kernel,
        out_type=jax.ShapeDtypeStruct((B, C, N), jnp.float32),
        mesh=mesh,
        compiler_params=pltpu.CompilerParams(
            needs_layout_passes=False, use_tc_tiling_on_sc=True),
        scratch_types=[
            pltpu.VMEM((N * 3,), jnp.float32),
            pltpu.VMEM((2, _L), jnp.float32),
            pltpu.VMEM((3, N), jnp.int32),
            pltpu.VMEM((4, N), jnp.float32),
            pltpu.VMEM((H, W), jnp.float32),
            pltpu.VMEM((N,), jnp.float32),
            pltpu.VMEM((N,), jnp.float32),
            pltpu.SemaphoreType.DMA,
            pltpu.SemaphoreType.DMA,
            pltpu.SemaphoreType.DMA,
        ],
    )
    def k(bev_hbm, kp_hbm, consts_hbm, out_hbm, kp_v, consts_v, idx_v, wgt_v,
          img, out0, out1, si, so0, so1):
        wid = lax.axis_index("s") * _NC + lax.axis_index("c")
        b = wid // w_per_b
        c0 = (wid % w_per_b) * per_w

        pltpu.async_copy(bev_hbm.at[pl.ds((b * C + c0) * H, H)], img, si)
        pltpu.sync_copy(kp_hbm.at[b], kp_v)
        pltpu.sync_copy(consts_hbm, consts_v)
        stride_v = consts_v[0, :]
        scale_v = consts_v[1, :]
        lane3 = lax.iota(jnp.int32, _L) * 3

        def prep_body(g2, carry):
            for u in range(2):
                g = g2 * 2 + u
                pos = lane3 + g * (3 * _L)
                xs = plsc.load_gather(kp_v, [pos])
                ys = plsc.load_gather(kp_v, [pos + 1])
                x = (xs - 0.0) / jnp.float32(0.05) / stride_v
                y = (ys - jnp.float32(-40.0)) / jnp.float32(0.05) / stride_v
                x0t = x.astype(jnp.int32)
                y0t = y.astype(jnp.int32)
                x0c = jnp.clip(x0t, 0, W - 1)
                x1c = jnp.clip(x0t + 1, 0, W - 1)
                y0c = jnp.clip(y0t, 0, H - 1)
                y1c = jnp.clip(y0t + 1, 0, H - 1)
                x0f = x0c.astype(jnp.float32)
                x1f = x1c.astype(jnp.float32)
                y0f = y0c.astype(jnp.float32)
                y1f = y1c.astype(jnp.float32)
                gx = x1f - x
                fx = x - x0f
                gy = y1f - y
                fy = y - y0f
                s = pl.ds(g * _L, _L)
                idx_v[0, s] = y0c
                idx_v[1, s] = x0c
                idx_v[2, s] = (y1c - y0c) * 2 + (x1c - x0c)
                wgt_v[0, s] = gx * gy * scale_v
                wgt_v[1, s] = gx * fy * scale_v
                wgt_v[2, s] = fx * gy * scale_v
                wgt_v[3, s] = fx * fy * scale_v
            return carry

        lax.fori_loop(0, n_grp // 2, prep_body, 0)

        obufs = ((out0, so0), (out1, so1))

        def chan_body(sidx, carry):
            for u in range(2):
                outb, so = obufs[u]
                c = c0 + sidx * 2 + u
                pltpu.make_async_copy(
                    bev_hbm.at[pl.ds((b * C + c) * H, H)], img, si).wait()

                @pl.when(sidx * 2 + u >= 2)
                def _wait_prev_store():
                    pltpu.make_async_copy(outb, out_hbm.at[b, c], so).wait()

                def grp_body(g4, carry2):
                    for v in range(4):
                        g = g4 * 4 + v
                        s = pl.ds(g * _L, _L)
                        iy0 = idx_v[0, s]
                        ix0 = idx_v[1, s]
                        code = idx_v[2, s]
                        dy = code >> 1
                        dx = code & 1
                        iy1 = iy0 + dy
                        ix1 = ix0 + dx
                        acc = plsc.load_gather(img, [iy0, ix0]) * wgt_v[0, s]
                        acc = acc + plsc.load_gather(img, [iy1, ix0]) * wgt_v[1, s]
                        acc = acc + plsc.load_gather(img, [iy0, ix1]) * wgt_v[2, s]
                        acc = acc + plsc.load_gather(img, [iy1, ix1]) * wgt_v[3, s]
                        outb[s] = acc
                    return carry2

                lax.fori_loop(0, n_grp // 4, grp_body, 0)

                @pl.when(c + 1 < c0 + per_w)
                def _prefetch_next():
                    pltpu.async_copy(
                        bev_hbm.at[pl.ds((b * C + c + 1) * H, H)], img, si)

                pltpu.async_copy(outb, out_hbm.at[b, c], so)
            return carry

        lax.fori_loop(0, per_w // 2, chan_body, 0)
        pltpu.make_async_copy(out0, out_hbm.at[b, c0], so0).wait()
        pltpu.make_async_copy(out1, out_hbm.at[b, c0], so1).wait()

    return k(bev2d, kp_flat, consts)


def kernel(keypoints, bev_features, batch_size, bev_stride):
    B, N, _ = keypoints.shape
    _, C, H, W = bev_features.shape
    stride_f = jnp.asarray(bev_stride, jnp.float32)
    scale_f = jnp.asarray(batch_size, jnp.float32) / B
    consts = jnp.stack([jnp.full((_L,), 1.0, jnp.float32) * stride_f,
                        jnp.full((_L,), 1.0, jnp.float32) * scale_f])
    kp_flat = keypoints.reshape(B, N * 3)
    bev2d = bev_features.reshape(B * C * H, W)
    out_bcn = _bilinear_gather(bev2d, kp_flat, consts, B, C, N, H, W)
    out = jnp.transpose(out_bcn, (0, 2, 1))
    return out


# speedup vs baseline: 1.8638x; 1.5043x over previous
import functools

import jax
import jax.numpy as jnp
from jax import lax
from jax.experimental import pallas as pl
from jax.experimental.pallas import tpu as pltpu
from jax.experimental.pallas import tpu_sc as plsc

_L = 16
_NC = 2
_NS = 16


def _bilinear_gather(bev_flat, kp_flat, consts, B, C, N, HW, H, W):
    NW = _NC * _NS
    per_w = (B * C) // NW
    w_per_b = NW // B
    n_grp = N // _L
    mesh = plsc.VectorSubcoreMesh(
        core_axis_name="c", subcore_axis_name="s",
        num_cores=_NC, num_subcores=_NS)

    @functools.partial(
        pl.kernel,
        out_type=jax.ShapeDtypeStruct((B, C, N), jnp.float32),
        mesh=mesh,
        compiler_params=pltpu.CompilerParams(needs_layout_passes=False),
        scratch_types=[
            pltpu.VMEM((N * 3,), jnp.float32),
            pltpu.VMEM((2, _L), jnp.float32),
            pltpu.VMEM((N,), jnp.int32),
            pltpu.VMEM((4, N), jnp.float32),
            pltpu.VMEM((HW,), jnp.float32),
            pltpu.VMEM((HW,), jnp.float32),
            pltpu.VMEM((N,), jnp.float32),
            pltpu.VMEM((N,), jnp.float32),
            pltpu.SemaphoreType.DMA,
            pltpu.SemaphoreType.DMA,
            pltpu.SemaphoreType.DMA,
            pltpu.SemaphoreType.DMA,
        ],
    )
    def k(bev_hbm, kp_hbm, consts_hbm, out_hbm, kp_v, consts_v, idx_v, wgt_v,
          img0, img1, out0, out1, si0, si1, so0, so1):
        wid = lax.axis_index("s") * _NC + lax.axis_index("c")
        b = wid // w_per_b
        c0 = (wid % w_per_b) * per_w

        pltpu.async_copy(bev_hbm.at[b, c0], img0, si0)
        pltpu.async_copy(bev_hbm.at[b, c0 + 1], img1, si1)
        pltpu.sync_copy(kp_hbm.at[b], kp_v)
        pltpu.sync_copy(consts_hbm, consts_v)
        stride_v = consts_v[0, :]
        scale_v = consts_v[1, :]
        lane3 = lax.iota(jnp.int32, _L) * 3

        def prep_body(g2, carry):
            for u in range(2):
                g = g2 * 2 + u
                pos = lane3 + g * (3 * _L)
                xs = plsc.load_gather(kp_v, [pos])
                ys = plsc.load_gather(kp_v, [pos + 1])
                x = (xs - 0.0) / jnp.float32(0.05) / stride_v
                y = (ys - jnp.float32(-40.0)) / jnp.float32(0.05) / stride_v
                x0t = x.astype(jnp.int32)
                y0t = y.astype(jnp.int32)
                x0c = jnp.clip(x0t, 0, W - 1)
                x1c = jnp.clip(x0t + 1, 0, W - 1)
                y0c = jnp.clip(y0t, 0, H - 1)
                y1c = jnp.clip(y0t + 1, 0, H - 1)
                x0f = x0c.astype(jnp.float32)
                x1f = x1c.astype(jnp.float32)
                y0f = y0c.astype(jnp.float32)
                y1f = y1c.astype(jnp.float32)
                gx = x1f - x
                fx = x - x0f
                gy = y1f - y
                fy = y - y0f
                s = pl.ds(g * _L, _L)
                idx_v[s] = (y0c * W + x0c) + (
                    ((y1c - y0c) * W + (x1c - x0c)) << 16)
                wgt_v[0, s] = gx * gy * scale_v
                wgt_v[1, s] = gx * fy * scale_v
                wgt_v[2, s] = fx * gy * scale_v
                wgt_v[3, s] = fx * fy * scale_v
            return carry

        lax.fori_loop(0, n_grp // 2, prep_body, 0)

        bufs = ((img0, out0, si0, so0), (img1, out1, si1, so1))

        def chan_body(sidx, carry):
            for u in range(2):
                img, outb, si, so = bufs[u]
                c = c0 + sidx * 2 + u
                pltpu.make_async_copy(bev_hbm.at[b, c], img, si).wait()

                @pl.when(sidx >= 1)
                def _wait_prev_store():
                    pltpu.make_async_copy(outb, out_hbm.at[b, c], so).wait()

                def grp_body(g8, carry2):
                    for v in range(8):
                        g = g8 * 8 + v
                        s = pl.ds(g * _L, _L)
                        word = idx_v[s]
                        ia = word & 0xFFFF
                        code = word >> 16
                        acc = plsc.load_gather(img, [ia]) * wgt_v[0, s]
                        acc = acc + plsc.load_gather(img, [ia + (code & -2)]) * wgt_v[1, s]
                        acc = acc + plsc.load_gather(img, [ia + (code & 1)]) * wgt_v[2, s]
                        acc = acc + plsc.load_gather(img, [ia + code]) * wgt_v[3, s]
                        outb[s] = acc
                    return carry2

                lax.fori_loop(0, n_grp // 8, grp_body, 0)
                pltpu.async_copy(outb, out_hbm.at[b, c], so)

                @pl.when(c + 2 < c0 + per_w)
                def _prefetch_next():
                    pltpu.async_copy(bev_hbm.at[b, c + 2], img, si)
            return carry

        lax.fori_loop(0, per_w // 2, chan_body, 0)
        pltpu.make_async_copy(out0, out_hbm.at[b, c0], so0).wait()
        pltpu.make_async_copy(out1, out_hbm.at[b, c0], so1).wait()

    return k(bev_flat, kp_flat, consts)


def kernel(keypoints, bev_features, batch_size, bev_stride):
    B, N, _ = keypoints.shape
    _, C, H, W = bev_features.shape
    stride_f = jnp.asarray(bev_stride, jnp.float32)
    scale_f = jnp.asarray(batch_size, jnp.float32) / B
    consts = jnp.stack([jnp.full((_L,), 1.0, jnp.float32) * stride_f,
                        jnp.full((_L,), 1.0, jnp.float32) * scale_f])
    kp_flat = keypoints.reshape(B, N * 3)
    bev_flat = bev_features.reshape(B, C, H * W)
    out_bcn = _bilinear_gather(bev_flat, kp_flat, consts, B, C, N, H * W, H, W)
    return jnp.transpose(out_bcn, (0, 2, 1))
